# baseline (speedup 1.0000x reference)
import functools

import jax
import jax.numpy as jnp
from jax import lax
from jax.experimental import pallas as pl
from jax.experimental.pallas import tpu as pltpu
from jax.experimental.pallas import tpu_sc as plsc

N, E, D = 10000, 320000, 128
NC, NS, L = 2, 16, 16
NW = NC * NS
C = 128
CPT = ((E + NW * C - 1) // (NW * C) + 1) // 2 * 2
NCHUNK = NW * CPT
EPAD = NCHUNK * C
RPT = 640
NPAD = NS * RPT
ZR = 32
K = 16
NB = CPT // K

@functools.cache
def _build_segsum():
    mesh = plsc.VectorSubcoreMesh(
        core_axis_name="c", subcore_axis_name="s",
        num_cores=NC, num_subcores=NS)
    return functools.partial(
        pl.kernel,
        mesh=mesh,
        out_type=jax.ShapeDtypeStruct((NC, NPAD, D), jnp.float32),
        scratch_types=[
            pltpu.VMEM((2, K, 2, C), jnp.int32),
            pltpu.VMEM((2, C, D), jnp.float32),
            pltpu.VMEM((ZR, D), jnp.float32),
            pltpu.VMEM_SHARED((NPAD, D), jnp.float32),
            pltpu.SemaphoreType.DMA,
            pltpu.SemaphoreType.DMA,
            pltpu.SemaphoreType.DMA,
            pltpu.SemaphoreType.DMA,
            pltpu.SemaphoreType.DMA,
            pltpu.SemaphoreType.DMA,
        ],
    )(_segsum_body)


H = 2
CH = C // H


def _segsum_body(table_hbm, idx_hbm, out_hbm,
                 idx_v, rows_v, zbuf, agg_sh, sem_z, sem_i,
                 sem_g0, sem_g1, sem_g2, sem_g3):
    c = lax.axis_index("c")
    s = lax.axis_index("s")
    wid = c * NS + s
    gsems = ((sem_g0, sem_g1), (sem_g2, sem_g3))
    tbase = wid * CPT

    def _zero_row(i, carry):
        for kk in range(D // L):
            zbuf[i, pl.ds(kk * L, L)] = jnp.zeros((L,), jnp.float32)
        return carry

    lax.fori_loop(0, ZR, _zero_row, 0)
    for t in range(RPT // ZR):
        pltpu.async_copy(zbuf, agg_sh.at[pl.ds(s * RPT + t * ZR, ZR)], sem_z)

    pltpu.sync_copy(idx_hbm.at[pl.ds(tbase, K)], idx_v.at[0])
    for b in range(2):
        for h in range(H):
            pltpu.async_copy(
                table_hbm.at[idx_v.at[0, b, 0, pl.ds(h * CH, CH)]],
                rows_v.at[b, pl.ds(h * CH, CH)], gsems[b][h])
    pltpu.async_copy(idx_hbm.at[pl.ds(tbase + K, K)], idx_v.at[1], sem_i)

    for t in range(RPT // ZR):
        pltpu.make_async_copy(zbuf, agg_sh.at[pl.ds(0, ZR)], sem_z).wait()
    plsc.subcore_barrier()

    def _pair(g, carry):
        for b in range(2):
            k = 2 * g + b
            blk = k // K
            off = k - blk * K
            slot = lax.rem(blk, 2)
            for h in range(H):
                pltpu.make_async_copy(
                    table_hbm.at[idx_v.at[0, 0, 0, pl.ds(0, CH)]],
                    rows_v.at[b, pl.ds(h * CH, CH)], gsems[b][h]).wait()
            pltpu.sync_copy(rows_v.at[b], agg_sh.at[idx_v.at[slot, off, 1]],
                            add=True)
            if b == 0:
                @pl.when(jnp.logical_and(off == 0, blk >= 1))
                def _():
                    nblk = jnp.minimum(blk + 1, NB - 1)
                    pltpu.async_copy(idx_hbm.at[pl.ds(tbase + nblk * K, K)],
                                     idx_v.at[lax.rem(blk + 1, 2)], sem_i)

                @pl.when(off == K - 2)
                def _():
                    pltpu.make_async_copy(idx_hbm.at[pl.ds(tbase, K)],
                                          idx_v.at[0], sem_i).wait()
            kn = jnp.minimum(k + 2, CPT - 1)
            nb2 = kn // K
            for h in range(H):
                pltpu.async_copy(
                    table_hbm.at[
                        idx_v.at[lax.rem(nb2, 2), kn - nb2 * K, 0,
                                 pl.ds(h * CH, CH)]],
                    rows_v.at[b, pl.ds(h * CH, CH)], gsems[b][h])
        return carry

    lax.fori_loop(0, CPT // 2, _pair, 0)
    for b in range(2):
        for h in range(H):
            pltpu.make_async_copy(
                table_hbm.at[idx_v.at[0, 0, 0, pl.ds(0, CH)]],
                rows_v.at[b, pl.ds(h * CH, CH)], gsems[b][h]).wait()

    plsc.subcore_barrier()
    pltpu.sync_copy(agg_sh.at[pl.ds(s * RPT, RPT)],
                    out_hbm.at[c, pl.ds(s * RPT, RPT)])


_BN = 2000


def _mlp_body(eps_ref, x_ref, agg_ref, wa_ref, ba_ref, wb_ref, bb_ref, o_ref):
    h = x_ref[...] * (1.0 + eps_ref[0, 0]) + agg_ref[0] + agg_ref[1]
    h = jnp.dot(h, wa_ref[...], preferred_element_type=jnp.float32) + ba_ref[...]
    h = jnp.maximum(h, 0.0)
    o_ref[...] = (
        jnp.dot(h, wb_ref[...], preferred_element_type=jnp.float32) + bb_ref[...]
    )


_mlp = pl.pallas_call(
    _mlp_body,
    grid=(N // _BN,),
    in_specs=[
        pl.BlockSpec(memory_space=pltpu.SMEM),
        pl.BlockSpec((_BN, D), lambda i: (i, 0)),
        pl.BlockSpec((NC, _BN, D), lambda i: (0, i, 0)),
        pl.BlockSpec((D, D), lambda i: (0, 0)),
        pl.BlockSpec((1, D), lambda i: (0, 0)),
        pl.BlockSpec((D, D), lambda i: (0, 0)),
        pl.BlockSpec((1, D), lambda i: (0, 0)),
    ],
    out_specs=pl.BlockSpec((_BN, D), lambda i: (i, 0)),
    out_shape=jax.ShapeDtypeStruct((N, D), jnp.float32),
)


def _gin_layer(x, idx2d, wa, ba, wb, bb, eps):
    agg = _build_segsum()(x, idx2d)
    return _mlp(eps.reshape(1, 1), x, agg, wa, ba.reshape(1, D),
                wb, bb.reshape(1, D))


def kernel(x, edge_index, W1a, b1a, W1b, b1b, eps1, W2a, b2a, W2b, b2b, eps2):
    pad = EPAD - E
    pad_src = jnp.arange(pad, dtype=jnp.int32) % N
    pad_dst = N + (jnp.arange(pad, dtype=jnp.int32) % (NPAD - N))
    src2d = jnp.concatenate([edge_index[0], pad_src]).reshape(NCHUNK, C)
    dst2d = jnp.concatenate([edge_index[1], pad_dst]).reshape(NCHUNK, C)
    idx2d = jnp.stack([src2d, dst2d], axis=1)
    h = _gin_layer(x, idx2d, W1a, b1a, W1b, b1b, eps1)
    h = _gin_layer(h, idx2d, W2a, b2a, W2b, b2b, eps2)
    return h


# speedup vs baseline: 1.0952x; 1.0952x over previous
import functools

import jax
import jax.numpy as jnp
from jax import lax
from jax.experimental import pallas as pl
from jax.experimental.pallas import tpu as pltpu
from jax.experimental.pallas import tpu_sc as plsc

N, E, D = 10000, 320000, 128
NC, NS, L = 2, 16, 16
NW = NC * NS
C = 128
CPT = ((E + NW * C - 1) // (NW * C) + 1) // 2 * 2
NCHUNK = NW * CPT
EPAD = NCHUNK * C
RPT = 640
NPAD = NS * RPT
ZR = 32
K = 16
NB = CPT // K

@functools.cache
def _build_segsum():
    mesh = plsc.VectorSubcoreMesh(
        core_axis_name="c", subcore_axis_name="s",
        num_cores=NC, num_subcores=NS)
    return functools.partial(
        pl.kernel,
        mesh=mesh,
        out_type=jax.ShapeDtypeStruct((NC, NPAD, D), jnp.float32),
        scratch_types=[
            pltpu.VMEM((2, K, 2, C), jnp.int32),
            pltpu.VMEM((2, C, D), jnp.float32),
            pltpu.VMEM((ZR, D), jnp.float32),
            pltpu.VMEM_SHARED((NPAD, D), jnp.float32),
            pltpu.SemaphoreType.DMA,
            pltpu.SemaphoreType.DMA,
            pltpu.SemaphoreType.DMA,
            pltpu.SemaphoreType.DMA,
            pltpu.SemaphoreType.DMA,
            pltpu.SemaphoreType.DMA,
        ],
    )(_segsum_body)


HC = C // 2


def _segsum_body(table_hbm, idx_hbm, out_hbm,
                 idx_v, rows_v, zbuf, agg_sh, sem_z, sem_i,
                 sem_g0, sem_g1, sem_g2, sem_g3):
    c = lax.axis_index("c")
    s = lax.axis_index("s")
    wid = c * NS + s
    gsems = ((sem_g0, sem_g1), (sem_g2, sem_g3))
    tbase = wid * CPT

    def _zero_row(i, carry):
        for kk in range(D // L):
            zbuf[i, pl.ds(kk * L, L)] = jnp.zeros((L,), jnp.float32)
        return carry

    lax.fori_loop(0, ZR, _zero_row, 0)
    for t in range(RPT // ZR):
        pltpu.async_copy(zbuf, agg_sh.at[pl.ds(s * RPT + t * ZR, ZR)], sem_z)

    pltpu.sync_copy(idx_hbm.at[pl.ds(tbase, K)], idx_v.at[0])
    for b in range(2):
        for h in range(2):
            pltpu.async_copy(
                table_hbm.at[idx_v.at[0, b, 0, pl.ds(h * HC, HC)]],
                rows_v.at[b, pl.ds(h * HC, HC)], gsems[b][h])
    pltpu.async_copy(idx_hbm.at[pl.ds(tbase + K, K)], idx_v.at[1], sem_i)

    for t in range(RPT // ZR):
        pltpu.make_async_copy(zbuf, agg_sh.at[pl.ds(0, ZR)], sem_z).wait()
    plsc.subcore_barrier()

    def _pair(g, carry):
        for b in range(2):
            k = 2 * g + b
            blk = k // K
            off = k - blk * K
            slot = lax.rem(blk, 2)
            if b == 0:
                @pl.when(jnp.logical_and(off == 0, blk >= 1))
                def _():
                    nblk = jnp.minimum(blk + 1, NB - 1)
                    pltpu.async_copy(idx_hbm.at[pl.ds(tbase + nblk * K, K)],
                                     idx_v.at[lax.rem(blk + 1, 2)], sem_i)

                @pl.when(off == K - 2)
                def _():
                    pltpu.make_async_copy(idx_hbm.at[pl.ds(tbase, K)],
                                          idx_v.at[0], sem_i).wait()
            kn = jnp.minimum(k + 2, CPT - 1)
            nb2 = kn // K
            for h in range(2):
                pltpu.make_async_copy(
                    table_hbm.at[idx_v.at[0, 0, 0, pl.ds(0, HC)]],
                    rows_v.at[b, pl.ds(h * HC, HC)], gsems[b][h]).wait()
            for h in range(2):
                pltpu.sync_copy(
                    rows_v.at[b, pl.ds(h * HC, HC)],
                    agg_sh.at[idx_v.at[slot, off, 1, pl.ds(h * HC, HC)]],
                    add=True)
                pltpu.async_copy(
                    table_hbm.at[idx_v.at[lax.rem(nb2, 2), kn - nb2 * K, 0,
                                          pl.ds(h * HC, HC)]],
                    rows_v.at[b, pl.ds(h * HC, HC)], gsems[b][h])
        return carry

    lax.fori_loop(0, CPT // 2, _pair, 0)
    for b in range(2):
        for h in range(2):
            pltpu.make_async_copy(
                table_hbm.at[idx_v.at[0, 0, 0, pl.ds(0, HC)]],
                rows_v.at[b, pl.ds(h * HC, HC)], gsems[b][h]).wait()

    plsc.subcore_barrier()
    pltpu.sync_copy(agg_sh.at[pl.ds(s * RPT, RPT)],
                    out_hbm.at[c, pl.ds(s * RPT, RPT)])


_BN = 2000


def _mlp_body(eps_ref, x_ref, agg_ref, wa_ref, ba_ref, wb_ref, bb_ref, o_ref):
    h = x_ref[...] * (1.0 + eps_ref[0, 0]) + agg_ref[0] + agg_ref[1]
    h = jnp.dot(h, wa_ref[...], preferred_element_type=jnp.float32) + ba_ref[...]
    h = jnp.maximum(h, 0.0)
    o_ref[...] = (
        jnp.dot(h, wb_ref[...], preferred_element_type=jnp.float32) + bb_ref[...]
    )


_mlp = pl.pallas_call(
    _mlp_body,
    grid=(N // _BN,),
    in_specs=[
        pl.BlockSpec(memory_space=pltpu.SMEM),
        pl.BlockSpec((_BN, D), lambda i: (i, 0)),
        pl.BlockSpec((NC, _BN, D), lambda i: (0, i, 0)),
        pl.BlockSpec((D, D), lambda i: (0, 0)),
        pl.BlockSpec((1, D), lambda i: (0, 0)),
        pl.BlockSpec((D, D), lambda i: (0, 0)),
        pl.BlockSpec((1, D), lambda i: (0, 0)),
    ],
    out_specs=pl.BlockSpec((_BN, D), lambda i: (i, 0)),
    out_shape=jax.ShapeDtypeStruct((N, D), jnp.float32),
)


def _gin_layer(x, idx2d, wa, ba, wb, bb, eps):
    agg = _build_segsum()(x, idx2d)
    return _mlp(eps.reshape(1, 1), x, agg, wa, ba.reshape(1, D),
                wb, bb.reshape(1, D))


def kernel(x, edge_index, W1a, b1a, W1b, b1b, eps1, W2a, b2a, W2b, b2b, eps2):
    pad = EPAD - E
    pad_src = jnp.arange(pad, dtype=jnp.int32) % N
    pad_dst = N + (jnp.arange(pad, dtype=jnp.int32) % (NPAD - N))
    src2d = jnp.concatenate([edge_index[0], pad_src]).reshape(NCHUNK, C)
    dst2d = jnp.concatenate([edge_index[1], pad_dst]).reshape(NCHUNK, C)
    idx2d = jnp.stack([src2d, dst2d], axis=1)
    h = _gin_layer(x, idx2d, W1a, b1a, W1b, b1b, eps1)
    h = _gin_layer(h, idx2d, W2a, b2a, W2b, b2b, eps2)
    return h


# speedup vs baseline: 1.1337x; 1.0351x over previous
import functools

import jax
import jax.numpy as jnp
from jax import lax
from jax.experimental import pallas as pl
from jax.experimental.pallas import tpu as pltpu
from jax.experimental.pallas import tpu_sc as plsc

N, E, D = 10000, 320000, 128
NC, NS, L = 2, 16, 16
NW = NC * NS
C = 128
CPT = ((E + NW * C - 1) // (NW * C) + 1) // 2 * 2
NCHUNK = NW * CPT
EPAD = NCHUNK * C
RPT = 640
NPAD = NS * RPT
ZR = 32
K = 16
NB = CPT // K

@functools.cache
def _build_segsum():
    mesh = plsc.VectorSubcoreMesh(
        core_axis_name="c", subcore_axis_name="s",
        num_cores=NC, num_subcores=NS)
    return functools.partial(
        pl.kernel,
        mesh=mesh,
        out_type=jax.ShapeDtypeStruct((NC, NPAD, D), jnp.float32),
        scratch_types=[
            pltpu.VMEM((2, K, 2, C), jnp.int32),
            pltpu.VMEM((2, C, D), jnp.float32),
            pltpu.VMEM((ZR, D), jnp.float32),
            pltpu.VMEM_SHARED((NPAD, D), jnp.float32),
            pltpu.SemaphoreType.DMA,
            pltpu.SemaphoreType.DMA,
            pltpu.SemaphoreType.DMA,
            pltpu.SemaphoreType.DMA,
            pltpu.SemaphoreType.DMA,
            pltpu.SemaphoreType.DMA,
            pltpu.SemaphoreType.DMA,
            pltpu.SemaphoreType.DMA,
            pltpu.SemaphoreType.DMA,
            pltpu.SemaphoreType.DMA,
        ],
    )(_segsum_body)


SPLIT = 4
HC = C // SPLIT


def _segsum_body(table_hbm, idx_hbm, out_hbm,
                 idx_v, rows_v, zbuf, agg_sh, sem_z, sem_i,
                 sem_g0, sem_g1, sem_g2, sem_g3,
                 sem_g4, sem_g5, sem_g6, sem_g7):
    c = lax.axis_index("c")
    s = lax.axis_index("s")
    wid = c * NS + s
    gsems = ((sem_g0, sem_g1, sem_g2, sem_g3),
             (sem_g4, sem_g5, sem_g6, sem_g7))
    tbase = wid * CPT

    def _zero_row(i, carry):
        for kk in range(D // L):
            zbuf[i, pl.ds(kk * L, L)] = jnp.zeros((L,), jnp.float32)
        return carry

    lax.fori_loop(0, ZR, _zero_row, 0)
    for t in range(RPT // ZR):
        pltpu.async_copy(zbuf, agg_sh.at[pl.ds(s * RPT + t * ZR, ZR)], sem_z)

    pltpu.sync_copy(idx_hbm.at[pl.ds(tbase, K)], idx_v.at[0])
    for b in range(2):
        for h in range(SPLIT):
            pltpu.async_copy(
                table_hbm.at[idx_v.at[0, b, 0, pl.ds(h * HC, HC)]],
                rows_v.at[b, pl.ds(h * HC, HC)], gsems[b][h])
    pltpu.async_copy(idx_hbm.at[pl.ds(tbase + K, K)], idx_v.at[1], sem_i)

    for t in range(RPT // ZR):
        pltpu.make_async_copy(zbuf, agg_sh.at[pl.ds(0, ZR)], sem_z).wait()
    plsc.subcore_barrier()

    def _pair(g, carry):
        for b in range(2):
            k = 2 * g + b
            blk = k // K
            off = k - blk * K
            slot = lax.rem(blk, 2)
            if b == 0:
                @pl.when(jnp.logical_and(off == 0, blk >= 1))
                def _():
                    nblk = jnp.minimum(blk + 1, NB - 1)
                    pltpu.async_copy(idx_hbm.at[pl.ds(tbase + nblk * K, K)],
                                     idx_v.at[lax.rem(blk + 1, 2)], sem_i)

                @pl.when(off == K - 2)
                def _():
                    pltpu.make_async_copy(idx_hbm.at[pl.ds(tbase, K)],
                                          idx_v.at[0], sem_i).wait()
            kn = jnp.minimum(k + 2, CPT - 1)
            nb2 = kn // K
            for h in range(SPLIT):
                pltpu.make_async_copy(
                    table_hbm.at[idx_v.at[0, 0, 0, pl.ds(0, HC)]],
                    rows_v.at[b, pl.ds(h * HC, HC)], gsems[b][h]).wait()
            for h in range(SPLIT):
                pltpu.sync_copy(
                    rows_v.at[b, pl.ds(h * HC, HC)],
                    agg_sh.at[idx_v.at[slot, off, 1, pl.ds(h * HC, HC)]],
                    add=True)
                pltpu.async_copy(
                    table_hbm.at[idx_v.at[lax.rem(nb2, 2), kn - nb2 * K, 0,
                                          pl.ds(h * HC, HC)]],
                    rows_v.at[b, pl.ds(h * HC, HC)], gsems[b][h])
        return carry

    lax.fori_loop(0, CPT // 2, _pair, 0)
    for b in range(2):
        for h in range(SPLIT):
            pltpu.make_async_copy(
                table_hbm.at[idx_v.at[0, 0, 0, pl.ds(0, HC)]],
                rows_v.at[b, pl.ds(h * HC, HC)], gsems[b][h]).wait()

    plsc.subcore_barrier()
    pltpu.sync_copy(agg_sh.at[pl.ds(s * RPT, RPT)],
                    out_hbm.at[c, pl.ds(s * RPT, RPT)])


_BN = 2000


def _mlp_body(eps_ref, x_ref, agg_ref, wa_ref, ba_ref, wb_ref, bb_ref, o_ref):
    h = x_ref[...] * (1.0 + eps_ref[0, 0]) + agg_ref[0] + agg_ref[1]
    h = jnp.dot(h, wa_ref[...], preferred_element_type=jnp.float32) + ba_ref[...]
    h = jnp.maximum(h, 0.0)
    o_ref[...] = (
        jnp.dot(h, wb_ref[...], preferred_element_type=jnp.float32) + bb_ref[...]
    )


_mlp = pl.pallas_call(
    _mlp_body,
    grid=(N // _BN,),
    in_specs=[
        pl.BlockSpec(memory_space=pltpu.SMEM),
        pl.BlockSpec((_BN, D), lambda i: (i, 0)),
        pl.BlockSpec((NC, _BN, D), lambda i: (0, i, 0)),
        pl.BlockSpec((D, D), lambda i: (0, 0)),
        pl.BlockSpec((1, D), lambda i: (0, 0)),
        pl.BlockSpec((D, D), lambda i: (0, 0)),
        pl.BlockSpec((1, D), lambda i: (0, 0)),
    ],
    out_specs=pl.BlockSpec((_BN, D), lambda i: (i, 0)),
    out_shape=jax.ShapeDtypeStruct((N, D), jnp.float32),
)


def _gin_layer(x, idx2d, wa, ba, wb, bb, eps):
    agg = _build_segsum()(x, idx2d)
    return _mlp(eps.reshape(1, 1), x, agg, wa, ba.reshape(1, D),
                wb, bb.reshape(1, D))


def kernel(x, edge_index, W1a, b1a, W1b, b1b, eps1, W2a, b2a, W2b, b2b, eps2):
    pad = EPAD - E
    pad_src = jnp.arange(pad, dtype=jnp.int32) % N
    pad_dst = N + (jnp.arange(pad, dtype=jnp.int32) % (NPAD - N))
    src2d = jnp.concatenate([edge_index[0], pad_src]).reshape(NCHUNK, C)
    dst2d = jnp.concatenate([edge_index[1], pad_dst]).reshape(NCHUNK, C)
    idx2d = jnp.stack([src2d, dst2d], axis=1)
    h = _gin_layer(x, idx2d, W1a, b1a, W1b, b1b, eps1)
    h = _gin_layer(h, idx2d, W2a, b2a, W2b, b2b, eps2)
    return h
